# baseline (speedup 1.0000x reference)
import functools

import jax
import jax.numpy as jnp
from jax import lax
from jax.experimental import pallas as pl
from jax.experimental.pallas import tpu as pltpu
from jax.experimental.pallas import tpu_sc as plsc

N_ENTITY = 1000000
N_RELATION = 1000
BATCH = 16384
DIM = 64

NC = 2
NS = 16
L = 16
NW = NC * NS
PANEL = 128
N_PANEL_FULL = N_ENTITY // PANEL
N_PANELS = N_PANEL_FULL + 1
P_PER_W = (N_PANELS + NW - 1) // NW
RING = 5
MAXM = 2 * BATCH + L
B_PER_W = BATCH // NW
K2CHUNK = 64

_params = pltpu.CompilerParams(needs_layout_passes=False, use_tc_tiling_on_sc=True)
_mesh = dict(core_axis_name="c", subcore_axis_name="s", num_cores=NC, num_subcores=NS)


def _sweep_body(head_hbm, tail_hbm, emb_et_hbm, tail_panel_hbm, hm_hbm,
                idx_v, ents_v, bents_v, panels_v, orows_v,
                cnt_s, hist_s, start_s, pend_s,
                sems, osems, sem):
    wid = lax.axis_index("s") * NC + lax.axis_index("c")
    c0 = wid * P_PER_W
    c1 = jnp.minimum(c0 + P_PER_W, N_PANELS)
    e_lo = c0 * PANEL
    e_hi = jnp.minimum(c1 * PANEL, N_ENTITY)
    lane = lax.broadcasted_iota(jnp.int32, (L,), 0)

    n_my_panels = c1 - c0

    def fire(slot, ci):
        @pl.when(ci < n_my_panels)
        def _():
            c = c0 + ci

            @pl.when(c < N_PANEL_FULL)
            def _():
                off = pl.multiple_of(c * PANEL, PANEL)
                pltpu.async_copy(emb_et_hbm.at[:, pl.ds(off, PANEL)],
                                 panels_v.at[slot], sems[slot])

            @pl.when(c >= N_PANEL_FULL)
            def _():
                pltpu.async_copy(tail_panel_hbm, panels_v.at[slot], sems[slot])

    for slot in range(RING):
        fire(slot, jnp.int32(slot))

    cnt_s[0] = 0
    for slot in range(RING):
        pend_s[slot] = 0

    def scan_block(role, src_hbm):
        def blk(i, carry):
            pltpu.sync_copy(src_hbm.at[pl.ds(i * 2048, 2048)], idx_v)

            def chunk(j, carry2):
                e = idx_v[pl.ds(j * L, L)]
                m = (e >= e_lo) & (e < e_hi)
                npop = plsc.all_reduce_population_count(m)
                cnt = cnt_s[0]
                slot = (i * 2048 + j * L + role * BATCH) + lane
                packed = (jax.lax.shift_left((e >> 7) - c0, 22)
                          | jax.lax.shift_left(slot, 7) | (e & 127))
                plsc.store_compressed(ents_v.at[pl.ds(cnt, L)], packed, mask=m)
                cnt_s[0] = cnt + npop[0]
                return carry2

            return lax.fori_loop(0, 2048 // L, chunk, carry)

        lax.fori_loop(0, BATCH // 2048, blk, 0)

    scan_block(0, head_hbm)
    scan_block(1, tail_hbm)
    nmatch = cnt_s[0]

    def hzero(p, carry):
        hist_s[p] = 0
        return carry

    lax.fori_loop(0, P_PER_W, hzero, 0)

    def hcount(i, carry):
        ev = ents_v[pl.ds(i * L, L)]
        for k in range(L):
            @pl.when((i * L + k) < nmatch)
            def _():
                p = ev[k] >> 22
                hist_s[p] = hist_s[p] + 1
        return carry

    lax.fori_loop(0, (nmatch + L - 1) // L, hcount, 0)

    def prefix(p, acc):
        h = hist_s[p]
        start_s[p] = acc
        return acc + h

    lax.fori_loop(0, P_PER_W, prefix, 0)

    def bsort(i, carry):
        ev = ents_v[pl.ds(i * L, L)]
        for k in range(L):
            @pl.when((i * L + k) < nmatch)
            def _():
                p = ev[k] >> 22
                pos = start_s[p]
                onek = lane == k
                plsc.store_compressed(bents_v.at[pl.ds(pos, L)],
                                      jnp.full((L,), ev[k], jnp.int32), mask=onek)
                start_s[p] = pos + 1
        return carry

    lax.fori_loop(0, (nmatch + L - 1) // L, bsort, 0)

    def wave(v, carry):
        for slot in range(RING):
            ci = v * RING + slot

            @pl.when(ci < n_my_panels)
            def _():
                c = c0 + ci
                pltpu.make_async_copy(tail_panel_hbm, panels_v.at[slot],
                                      sems[slot]).wait()

                def drain(_, carry3):
                    pltpu.make_async_copy(orows_v.at[slot],
                                          hm_hbm.at[pl.ds(0, L)],
                                          osems[slot]).wait()
                    return carry3

                lax.fori_loop(0, pend_s[slot], drain, 0)
                s = jnp.where(c > c0, start_s[jnp.maximum(ci - 1, 0)], 0)
                e = start_s[ci]

                def mchunk(i, carry2):
                    m0 = s + i * L
                    valid = (m0 + lane) < e
                    pk = bents_v[pl.ds(m0, L)]
                    pk = jnp.where(valid, pk, jnp.full((L,), pk[0], jnp.int32))
                    col = pk & 127
                    mv = (pk >> 7) & 32767
                    smod = mv & 63
                    for d in range(DIM):
                        dsp = jnp.full((L,), d, jnp.int32)
                        vals = plsc.load_gather(panels_v.at[slot], [dsp, col])
                        plsc.store_scatter(orows_v.at[slot],
                                           [lane, (dsp + smod) & 63], vals)
                    pltpu.async_copy(orows_v.at[slot], hm_hbm.at[mv],
                                     osems[slot])
                    return carry2

                nchunks = jnp.maximum((e - s + L - 1) // L, 0)
                lax.fori_loop(0, nchunks, mchunk, 0)
                pend_s[slot] = nchunks
                fire(slot, ci + RING)
        return carry

    lax.fori_loop(0, (P_PER_W + RING - 1) // RING, wave, 0)

    for slot in range(RING):
        def fdrain(_, carry3):
            pltpu.make_async_copy(orows_v.at[slot], hm_hbm.at[pl.ds(0, L)],
                                  osems[slot]).wait()
            return carry3

        lax.fori_loop(0, pend_s[slot], fdrain, 0)


def _score_body(rel_hbm, hm_hbm, emb_rt_hbm, out_hbm,
                ridx_v, h_v, t_v, rt_v, out_v, sems2, sem):
    wid = lax.axis_index("s") * NC + lax.axis_index("c")
    base = wid * B_PER_W
    lane = lax.broadcasted_iota(jnp.int32, (L,), 0)
    npass = B_PER_W // K2CHUNK

    for p in range(8):
        off = pl.multiple_of(p * PANEL, PANEL)
        pltpu.async_copy(emb_rt_hbm.at[:, pl.ds(off, PANEL)],
                         rt_v.at[:, pl.ds(off, PANEL)], sem)

    def fire(slot, half):
        if half < npass:
            b0 = base + half * K2CHUNK
            pltpu.async_copy(rel_hbm.at[pl.ds(b0, K2CHUNK)],
                             ridx_v.at[pl.ds(slot * K2CHUNK, K2CHUNK)],
                             sems2[slot])
            pltpu.async_copy(hm_hbm.at[pl.ds(b0, K2CHUNK)], h_v.at[slot],
                             sems2[slot])
            pltpu.async_copy(hm_hbm.at[pl.ds(BATCH + b0, K2CHUNK)],
                             t_v.at[slot], sems2[slot])

    fire(0, 0)
    fire(1, 1)
    pltpu.make_async_copy(emb_rt_hbm, rt_v, sem).wait()

    for half in range(npass):
        slot = half % 2
        b0 = base + half * K2CHUNK
        pltpu.make_async_copy(rel_hbm.at[pl.ds(0, K2CHUNK)],
                              ridx_v.at[pl.ds(slot * K2CHUNK, K2CHUNK)],
                              sems2[slot]).wait()
        pltpu.make_async_copy(hm_hbm.at[pl.ds(0, K2CHUNK)], h_v.at[slot],
                              sems2[slot]).wait()
        pltpu.make_async_copy(hm_hbm.at[pl.ds(0, K2CHUNK)], t_v.at[slot],
                              sems2[slot]).wait()

        def group(g, carry):
            rows = g * L + lane
            rel = ridx_v[pl.ds(slot * K2CHUNK + g * L, L)]
            smod = (b0 + g * L + lane) & 63
            acc = jnp.zeros((L,), jnp.float32)
            for d in range(DIM):
                dsp = jnp.full((L,), d, jnp.int32)
                sk = (dsp + smod) & 63
                hh = plsc.load_gather(h_v.at[slot], [rows, sk])
                tt = plsc.load_gather(t_v.at[slot], [rows, sk])
                rr = plsc.load_gather(rt_v, [dsp, rel])
                acc = acc + hh * tt * rr
            out_v[pl.ds(half * K2CHUNK + g * L, L)] = acc
            return carry

        lax.fori_loop(0, K2CHUNK // L, group, 0)
        fire(slot, half + 2)

    pltpu.sync_copy(out_v, out_hbm.at[pl.ds(base, B_PER_W)])


@jax.jit
def kernel(head, tail, relation, emb_E, emb_R):
    head = head.astype(jnp.int32)
    tail = tail.astype(jnp.int32)
    relation = relation.astype(jnp.int32)
    emb_et = emb_E.T
    tail_panel = jnp.pad(emb_E[N_PANEL_FULL * PANEL:], ((0, 64), (0, 0))).T
    emb_rt = jnp.pad(emb_R, ((0, 1024 - N_RELATION), (0, 0))).T

    sweep = pl.kernel(
        _sweep_body,
        out_type=jax.ShapeDtypeStruct((2 * BATCH, PANEL), jnp.float32),
        mesh=plsc.VectorSubcoreMesh(**_mesh),
        compiler_params=_params,
        scratch_types=[
            pltpu.VMEM((2048,), jnp.int32),
            pltpu.VMEM((MAXM,), jnp.int32),
            pltpu.VMEM((MAXM,), jnp.int32),
            pltpu.VMEM((RING, DIM, PANEL), jnp.float32),
            pltpu.VMEM((RING, L, PANEL), jnp.float32),
            pltpu.SMEM((1,), jnp.int32),
            pltpu.SMEM((P_PER_W,), jnp.int32),
            pltpu.SMEM((P_PER_W,), jnp.int32),
            pltpu.SMEM((RING,), jnp.int32),
            [pltpu.SemaphoreType.DMA] * RING,
            [pltpu.SemaphoreType.DMA] * RING,
            pltpu.SemaphoreType.DMA,
        ],
    )
    hm = sweep(head, tail, emb_et, tail_panel)

    score = pl.kernel(
        _score_body,
        out_type=jax.ShapeDtypeStruct((BATCH,), jnp.float32),
        mesh=plsc.VectorSubcoreMesh(**_mesh),
        compiler_params=_params,
        scratch_types=[
            pltpu.VMEM((2 * K2CHUNK,), jnp.int32),
            pltpu.VMEM((2, K2CHUNK, PANEL), jnp.float32),
            pltpu.VMEM((2, K2CHUNK, PANEL), jnp.float32),
            pltpu.VMEM((DIM, 1024), jnp.float32),
            pltpu.VMEM((B_PER_W,), jnp.float32),
            [pltpu.SemaphoreType.DMA] * 2,
            pltpu.SemaphoreType.DMA,
        ],
    )
    return score(relation, hm, emb_rt)


# speedup vs baseline: 1.1498x; 1.1498x over previous
import functools

import jax
import jax.numpy as jnp
from jax import lax
from jax.experimental import pallas as pl
from jax.experimental.pallas import tpu as pltpu
from jax.experimental.pallas import tpu_sc as plsc

N_ENTITY = 1000000
N_RELATION = 1000
BATCH = 16384
DIM = 64

NC = 2
NS = 16
L = 16
NW = NC * NS
PANEL = 128
N_PANEL_FULL = N_ENTITY // PANEL
N_PANELS = N_PANEL_FULL + 1
P_PER_W = (N_PANELS + NW - 1) // NW
RING = 4
MAXM = 2 * BATCH + L
B_PER_W = BATCH // NW
K2CHUNK = 64

_params = pltpu.CompilerParams(needs_layout_passes=False, use_tc_tiling_on_sc=True)
_mesh = dict(core_axis_name="c", subcore_axis_name="s", num_cores=NC, num_subcores=NS)


def _sweep_body(head_hbm, tail_hbm, emb_et_hbm, tail_panel_hbm, hm_hbm,
                idx_v, ents_v, bents_v, panels_v, orows_v,
                cnt_s, hist_s, start_s, pend_s,
                sems, osems, sem):
    wid = lax.axis_index("s") * NC + lax.axis_index("c")
    c0 = wid * P_PER_W
    c1 = jnp.minimum(c0 + P_PER_W, N_PANELS)
    e_lo = c0 * PANEL
    e_hi = jnp.minimum(c1 * PANEL, N_ENTITY)
    lane = lax.broadcasted_iota(jnp.int32, (L,), 0)

    cnt_s[0] = 0
    for slot in range(RING):
        pend_s[slot] = 0

    def scan_block(role, src_hbm):
        def blk(i, carry):
            pltpu.sync_copy(src_hbm.at[pl.ds(i * 2048, 2048)], idx_v)

            def chunk(j, carry2):
                e = idx_v[pl.ds(j * L, L)]
                m = (e >= e_lo) & (e < e_hi)
                npop = plsc.all_reduce_population_count(m)
                cnt = cnt_s[0]
                slot = (i * 2048 + j * L + role * BATCH) + lane
                packed = (jax.lax.shift_left((e >> 7) - c0, 22)
                          | jax.lax.shift_left(slot, 7) | (e & 127))
                plsc.store_compressed(ents_v.at[pl.ds(cnt, L)], packed, mask=m)
                cnt_s[0] = cnt + npop[0]
                return carry2

            return lax.fori_loop(0, 2048 // L, chunk, carry)

        lax.fori_loop(0, BATCH // 2048, blk, 0)

    scan_block(0, head_hbm)
    scan_block(1, tail_hbm)
    nmatch = cnt_s[0]

    def hzero(p, carry):
        hist_s[p] = 0
        return carry

    lax.fori_loop(0, P_PER_W, hzero, 0)

    def hcount(i, carry):
        ev = ents_v[pl.ds(i * L, L)]
        for k in range(L):
            @pl.when((i * L + k) < nmatch)
            def _():
                p = ev[k] >> 22
                hist_s[p] = hist_s[p] + 1
        return carry

    lax.fori_loop(0, (nmatch + L - 1) // L, hcount, 0)

    def prefix(p, acc):
        h = hist_s[p]
        start_s[p] = acc
        return acc + h

    lax.fori_loop(0, P_PER_W, prefix, 0)

    def bsort(i, carry):
        ev = ents_v[pl.ds(i * L, L)]
        for k in range(L):
            @pl.when((i * L + k) < nmatch)
            def _():
                p = ev[k] >> 22
                pos = start_s[p]
                onek = lane == k
                plsc.store_compressed(bents_v.at[pl.ds(pos, L)],
                                      jnp.full((L,), ev[k], jnp.int32), mask=onek)
                start_s[p] = pos + 1
        return carry

    lax.fori_loop(0, (nmatch + L - 1) // L, bsort, 0)

    n_my_panels = c1 - c0

    def fire(slot, ci):
        @pl.when(ci < n_my_panels)
        def _():
            c = c0 + ci

            @pl.when(c < N_PANEL_FULL)
            def _():
                off = pl.multiple_of(c * PANEL, PANEL)
                pltpu.async_copy(emb_et_hbm.at[:, pl.ds(off, PANEL)],
                                 panels_v.at[slot], sems[slot])

            @pl.when(c >= N_PANEL_FULL)
            def _():
                pltpu.async_copy(tail_panel_hbm, panels_v.at[slot], sems[slot])

    for slot in range(RING):
        fire(slot, jnp.int32(slot))

    def wave(v, carry):
        for slot in range(RING):
            ci = v * RING + slot

            @pl.when(ci < n_my_panels)
            def _():
                c = c0 + ci
                pltpu.make_async_copy(tail_panel_hbm, panels_v.at[slot],
                                      sems[slot]).wait()

                def drain(_, carry3):
                    pltpu.make_async_copy(orows_v.at[slot],
                                          hm_hbm.at[pl.ds(0, L)],
                                          osems[slot]).wait()
                    return carry3

                lax.fori_loop(0, pend_s[slot], drain, 0)
                s = jnp.where(c > c0, start_s[jnp.maximum(ci - 1, 0)], 0)
                e = start_s[ci]

                def mchunk(i, carry2):
                    m0 = s + i * L
                    valid = (m0 + lane) < e
                    pk = bents_v[pl.ds(m0, L)]
                    pk = jnp.where(valid, pk, jnp.full((L,), pk[0], jnp.int32))
                    col = pk & 127
                    mv = (pk >> 7) & 32767
                    smod = mv & 63
                    for d in range(DIM):
                        dsp = jnp.full((L,), d, jnp.int32)
                        vals = plsc.load_gather(panels_v.at[slot], [dsp, col])
                        plsc.store_scatter(orows_v.at[slot],
                                           [lane, (dsp + smod) & 63], vals)
                    pltpu.async_copy(orows_v.at[slot], hm_hbm.at[mv],
                                     osems[slot])
                    return carry2

                nchunks = jnp.maximum((e - s + L - 1) // L, 0)
                lax.fori_loop(0, nchunks, mchunk, 0)
                pend_s[slot] = nchunks
                fire(slot, ci + RING)
        return carry

    lax.fori_loop(0, (P_PER_W + RING - 1) // RING, wave, 0)

    for slot in range(RING):
        def fdrain(_, carry3):
            pltpu.make_async_copy(orows_v.at[slot], hm_hbm.at[pl.ds(0, L)],
                                  osems[slot]).wait()
            return carry3

        lax.fori_loop(0, pend_s[slot], fdrain, 0)


def _score_body(rel_hbm, hm_hbm, emb_rt_hbm, out_hbm,
                ridx_v, h_v, t_v, rt_v, out_v, sems2, sem):
    wid = lax.axis_index("s") * NC + lax.axis_index("c")
    base = wid * B_PER_W
    lane = lax.broadcasted_iota(jnp.int32, (L,), 0)
    npass = B_PER_W // K2CHUNK

    for p in range(8):
        off = pl.multiple_of(p * PANEL, PANEL)
        pltpu.async_copy(emb_rt_hbm.at[:, pl.ds(off, PANEL)],
                         rt_v.at[:, pl.ds(off, PANEL)], sem)

    def fire(slot, half):
        if half < npass:
            b0 = base + half * K2CHUNK
            pltpu.async_copy(rel_hbm.at[pl.ds(b0, K2CHUNK)],
                             ridx_v.at[pl.ds(slot * K2CHUNK, K2CHUNK)],
                             sems2[slot])
            pltpu.async_copy(hm_hbm.at[pl.ds(b0, K2CHUNK)], h_v.at[slot],
                             sems2[slot])
            pltpu.async_copy(hm_hbm.at[pl.ds(BATCH + b0, K2CHUNK)],
                             t_v.at[slot], sems2[slot])

    fire(0, 0)
    fire(1, 1)
    pltpu.make_async_copy(emb_rt_hbm, rt_v, sem).wait()

    for half in range(npass):
        slot = half % 2
        b0 = base + half * K2CHUNK
        pltpu.make_async_copy(rel_hbm.at[pl.ds(0, K2CHUNK)],
                              ridx_v.at[pl.ds(slot * K2CHUNK, K2CHUNK)],
                              sems2[slot]).wait()
        pltpu.make_async_copy(hm_hbm.at[pl.ds(0, K2CHUNK)], h_v.at[slot],
                              sems2[slot]).wait()
        pltpu.make_async_copy(hm_hbm.at[pl.ds(0, K2CHUNK)], t_v.at[slot],
                              sems2[slot]).wait()

        def group(g, carry):
            rows = g * L + lane
            rel = ridx_v[pl.ds(slot * K2CHUNK + g * L, L)]
            smod = (b0 + g * L + lane) & 63
            acc = jnp.zeros((L,), jnp.float32)
            for d in range(DIM):
                dsp = jnp.full((L,), d, jnp.int32)
                sk = (dsp + smod) & 63
                hh = plsc.load_gather(h_v.at[slot], [rows, sk])
                tt = plsc.load_gather(t_v.at[slot], [rows, sk])
                rr = plsc.load_gather(rt_v, [dsp, rel])
                acc = acc + hh * tt * rr
            out_v[pl.ds(half * K2CHUNK + g * L, L)] = acc
            return carry

        lax.fori_loop(0, K2CHUNK // L, group, 0)
        fire(slot, half + 2)

    pltpu.sync_copy(out_v, out_hbm.at[pl.ds(base, B_PER_W)])


@jax.jit
def kernel(head, tail, relation, emb_E, emb_R):
    head = head.astype(jnp.int32)
    tail = tail.astype(jnp.int32)
    relation = relation.astype(jnp.int32)
    emb_et = emb_E.T
    tail_panel = jnp.pad(emb_E[N_PANEL_FULL * PANEL:], ((0, 64), (0, 0))).T
    emb_rt = jnp.pad(emb_R, ((0, 1024 - N_RELATION), (0, 0))).T

    sweep = pl.kernel(
        _sweep_body,
        out_type=jax.ShapeDtypeStruct((2 * BATCH, PANEL), jnp.float32),
        mesh=plsc.VectorSubcoreMesh(**_mesh),
        compiler_params=_params,
        scratch_types=[
            pltpu.VMEM((2048,), jnp.int32),
            pltpu.VMEM((MAXM,), jnp.int32),
            pltpu.VMEM((MAXM,), jnp.int32),
            pltpu.VMEM((RING, DIM, PANEL), jnp.float32),
            pltpu.VMEM((RING, L, PANEL), jnp.float32),
            pltpu.SMEM((1,), jnp.int32),
            pltpu.SMEM((P_PER_W,), jnp.int32),
            pltpu.SMEM((P_PER_W,), jnp.int32),
            pltpu.SMEM((RING,), jnp.int32),
            [pltpu.SemaphoreType.DMA] * RING,
            [pltpu.SemaphoreType.DMA] * RING,
            pltpu.SemaphoreType.DMA,
        ],
    )
    hm = sweep(head, tail, emb_et, tail_panel)

    score = pl.kernel(
        _score_body,
        out_type=jax.ShapeDtypeStruct((BATCH,), jnp.float32),
        mesh=plsc.VectorSubcoreMesh(**_mesh),
        compiler_params=_params,
        scratch_types=[
            pltpu.VMEM((2 * K2CHUNK,), jnp.int32),
            pltpu.VMEM((2, K2CHUNK, PANEL), jnp.float32),
            pltpu.VMEM((2, K2CHUNK, PANEL), jnp.float32),
            pltpu.VMEM((DIM, 1024), jnp.float32),
            pltpu.VMEM((B_PER_W,), jnp.float32),
            [pltpu.SemaphoreType.DMA] * 2,
            pltpu.SemaphoreType.DMA,
        ],
    )
    return score(relation, hm, emb_rt)


# speedup vs baseline: 1.1526x; 1.0025x over previous
import functools

import jax
import jax.numpy as jnp
from jax import lax
from jax.experimental import pallas as pl
from jax.experimental.pallas import tpu as pltpu
from jax.experimental.pallas import tpu_sc as plsc

N_ENTITY = 1000000
N_RELATION = 1000
BATCH = 16384
DIM = 64

NC = 2
NS = 16
L = 16
NW = NC * NS
PANEL = 128
N_PANEL_FULL = N_ENTITY // PANEL
N_PANELS = N_PANEL_FULL + 1
P_PER_W = (N_PANELS + NW - 1) // NW
RING = 4
MAXM = 2 * BATCH + L
B_PER_W = BATCH // NW
K2CHUNK = 64

_params = pltpu.CompilerParams(needs_layout_passes=False, use_tc_tiling_on_sc=True)
_mesh = dict(core_axis_name="c", subcore_axis_name="s", num_cores=NC, num_subcores=NS)


def _sweep_body(head_hbm, tail_hbm, emb_et_hbm, tail_panel_hbm, hm_hbm,
                idx_v, ents_v, bents_v, panels_v, orows_v,
                cnt_s, hist_s, start_s, pend_s,
                sems, osems, sem):
    wid = lax.axis_index("s") * NC + lax.axis_index("c")
    c0 = wid * P_PER_W
    c1 = jnp.minimum(c0 + P_PER_W, N_PANELS)
    e_lo = c0 * PANEL
    e_hi = jnp.minimum(c1 * PANEL, N_ENTITY)
    lane = lax.broadcasted_iota(jnp.int32, (L,), 0)

    cnt_s[0] = 0
    for slot in range(RING):
        pend_s[slot] = 0

    def scan_block(role, src_hbm):
        def blk(i, carry):
            pltpu.sync_copy(src_hbm.at[pl.ds(i * 2048, 2048)], idx_v)

            def chunk(j, carry2):
                e = idx_v[pl.ds(j * L, L)]
                m = (e >= e_lo) & (e < e_hi)
                npop = plsc.all_reduce_population_count(m)
                cnt = cnt_s[0]
                slot = (i * 2048 + j * L + role * BATCH) + lane
                packed = (jax.lax.shift_left((e >> 7) - c0, 22)
                          | jax.lax.shift_left(slot, 7) | (e & 127))
                plsc.store_compressed(ents_v.at[pl.ds(cnt, L)], packed, mask=m)
                cnt_s[0] = cnt + npop[0]
                return carry2

            return lax.fori_loop(0, 2048 // L, chunk, carry)

        lax.fori_loop(0, BATCH // 2048, blk, 0)

    scan_block(0, head_hbm)
    scan_block(1, tail_hbm)
    nmatch = cnt_s[0]

    n_my_panels = c1 - c0

    def fire(slot, ci):
        @pl.when(ci < n_my_panels)
        def _():
            c = c0 + ci

            @pl.when(c < N_PANEL_FULL)
            def _():
                off = pl.multiple_of(c * PANEL, PANEL)
                pltpu.async_copy(emb_et_hbm.at[:, pl.ds(off, PANEL)],
                                 panels_v.at[slot], sems[slot])

            @pl.when(c >= N_PANEL_FULL)
            def _():
                pltpu.async_copy(tail_panel_hbm, panels_v.at[slot], sems[slot])

    for slot in range(RING):
        fire(slot, jnp.int32(slot))


    def hzero(p, carry):
        hist_s[p] = 0
        return carry

    lax.fori_loop(0, P_PER_W, hzero, 0)

    def hcount(i, carry):
        ev = ents_v[pl.ds(i * L, L)]
        for k in range(L):
            @pl.when((i * L + k) < nmatch)
            def _():
                p = ev[k] >> 22
                hist_s[p] = hist_s[p] + 1
        return carry

    lax.fori_loop(0, (nmatch + L - 1) // L, hcount, 0)

    def prefix(p, acc):
        h = hist_s[p]
        start_s[p] = acc
        return acc + h

    lax.fori_loop(0, P_PER_W, prefix, 0)

    def bsort(i, carry):
        ev = ents_v[pl.ds(i * L, L)]
        for k in range(L):
            @pl.when((i * L + k) < nmatch)
            def _():
                p = ev[k] >> 22
                pos = start_s[p]
                onek = lane == k
                plsc.store_compressed(bents_v.at[pl.ds(pos, L)],
                                      jnp.full((L,), ev[k], jnp.int32), mask=onek)
                start_s[p] = pos + 1
        return carry

    lax.fori_loop(0, (nmatch + L - 1) // L, bsort, 0)

    def wave(v, carry):
        for slot in range(RING):
            ci = v * RING + slot

            @pl.when(ci < n_my_panels)
            def _():
                c = c0 + ci
                pltpu.make_async_copy(tail_panel_hbm, panels_v.at[slot],
                                      sems[slot]).wait()

                def drain(_, carry3):
                    pltpu.make_async_copy(orows_v.at[slot],
                                          hm_hbm.at[pl.ds(0, L)],
                                          osems[slot]).wait()
                    return carry3

                lax.fori_loop(0, pend_s[slot], drain, 0)
                s = jnp.where(c > c0, start_s[jnp.maximum(ci - 1, 0)], 0)
                e = start_s[ci]

                def mchunk(i, carry2):
                    m0 = s + i * L
                    valid = (m0 + lane) < e
                    pk = bents_v[pl.ds(m0, L)]
                    pk = jnp.where(valid, pk, jnp.full((L,), pk[0], jnp.int32))
                    col = pk & 127
                    mv = (pk >> 7) & 32767
                    smod = mv & 63
                    for d in range(DIM):
                        dsp = jnp.full((L,), d, jnp.int32)
                        vals = plsc.load_gather(panels_v.at[slot], [dsp, col])
                        plsc.store_scatter(orows_v.at[slot],
                                           [lane, (dsp + smod) & 63], vals)
                    pltpu.async_copy(orows_v.at[slot], hm_hbm.at[mv],
                                     osems[slot])
                    return carry2

                nchunks = jnp.maximum((e - s + L - 1) // L, 0)
                lax.fori_loop(0, nchunks, mchunk, 0)
                pend_s[slot] = nchunks
                fire(slot, ci + RING)
        return carry

    lax.fori_loop(0, (P_PER_W + RING - 1) // RING, wave, 0)

    for slot in range(RING):
        def fdrain(_, carry3):
            pltpu.make_async_copy(orows_v.at[slot], hm_hbm.at[pl.ds(0, L)],
                                  osems[slot]).wait()
            return carry3

        lax.fori_loop(0, pend_s[slot], fdrain, 0)


def _score_body(rel_hbm, hm_hbm, emb_rt_hbm, out_hbm,
                ridx_v, h_v, t_v, rt_v, out_v, sems2, sem):
    wid = lax.axis_index("s") * NC + lax.axis_index("c")
    base = wid * B_PER_W
    lane = lax.broadcasted_iota(jnp.int32, (L,), 0)
    npass = B_PER_W // K2CHUNK

    for p in range(8):
        off = pl.multiple_of(p * PANEL, PANEL)
        pltpu.async_copy(emb_rt_hbm.at[:, pl.ds(off, PANEL)],
                         rt_v.at[:, pl.ds(off, PANEL)], sem)

    def fire(slot, half):
        if half < npass:
            b0 = base + half * K2CHUNK
            pltpu.async_copy(rel_hbm.at[pl.ds(b0, K2CHUNK)],
                             ridx_v.at[pl.ds(slot * K2CHUNK, K2CHUNK)],
                             sems2[slot])
            pltpu.async_copy(hm_hbm.at[pl.ds(b0, K2CHUNK)], h_v.at[slot],
                             sems2[slot])
            pltpu.async_copy(hm_hbm.at[pl.ds(BATCH + b0, K2CHUNK)],
                             t_v.at[slot], sems2[slot])

    fire(0, 0)
    fire(1, 1)
    pltpu.make_async_copy(emb_rt_hbm, rt_v, sem).wait()

    for half in range(npass):
        slot = half % 2
        b0 = base + half * K2CHUNK
        pltpu.make_async_copy(rel_hbm.at[pl.ds(0, K2CHUNK)],
                              ridx_v.at[pl.ds(slot * K2CHUNK, K2CHUNK)],
                              sems2[slot]).wait()
        pltpu.make_async_copy(hm_hbm.at[pl.ds(0, K2CHUNK)], h_v.at[slot],
                              sems2[slot]).wait()
        pltpu.make_async_copy(hm_hbm.at[pl.ds(0, K2CHUNK)], t_v.at[slot],
                              sems2[slot]).wait()

        def group(g, carry):
            rows = g * L + lane
            rel = ridx_v[pl.ds(slot * K2CHUNK + g * L, L)]
            smod = (b0 + g * L + lane) & 63
            acc = jnp.zeros((L,), jnp.float32)
            for d in range(DIM):
                dsp = jnp.full((L,), d, jnp.int32)
                sk = (dsp + smod) & 63
                hh = plsc.load_gather(h_v.at[slot], [rows, sk])
                tt = plsc.load_gather(t_v.at[slot], [rows, sk])
                rr = plsc.load_gather(rt_v, [dsp, rel])
                acc = acc + hh * tt * rr
            out_v[pl.ds(half * K2CHUNK + g * L, L)] = acc
            return carry

        lax.fori_loop(0, K2CHUNK // L, group, 0)
        fire(slot, half + 2)

    pltpu.sync_copy(out_v, out_hbm.at[pl.ds(base, B_PER_W)])


@jax.jit
def kernel(head, tail, relation, emb_E, emb_R):
    head = head.astype(jnp.int32)
    tail = tail.astype(jnp.int32)
    relation = relation.astype(jnp.int32)
    emb_et = emb_E.T
    tail_panel = jnp.pad(emb_E[N_PANEL_FULL * PANEL:], ((0, 64), (0, 0))).T
    emb_rt = jnp.pad(emb_R, ((0, 1024 - N_RELATION), (0, 0))).T

    sweep = pl.kernel(
        _sweep_body,
        out_type=jax.ShapeDtypeStruct((2 * BATCH, PANEL), jnp.float32),
        mesh=plsc.VectorSubcoreMesh(**_mesh),
        compiler_params=_params,
        scratch_types=[
            pltpu.VMEM((2048,), jnp.int32),
            pltpu.VMEM((MAXM,), jnp.int32),
            pltpu.VMEM((MAXM,), jnp.int32),
            pltpu.VMEM((RING, DIM, PANEL), jnp.float32),
            pltpu.VMEM((RING, L, PANEL), jnp.float32),
            pltpu.SMEM((1,), jnp.int32),
            pltpu.SMEM((P_PER_W,), jnp.int32),
            pltpu.SMEM((P_PER_W,), jnp.int32),
            pltpu.SMEM((RING,), jnp.int32),
            [pltpu.SemaphoreType.DMA] * RING,
            [pltpu.SemaphoreType.DMA] * RING,
            pltpu.SemaphoreType.DMA,
        ],
    )
    hm = sweep(head, tail, emb_et, tail_panel)

    score = pl.kernel(
        _score_body,
        out_type=jax.ShapeDtypeStruct((BATCH,), jnp.float32),
        mesh=plsc.VectorSubcoreMesh(**_mesh),
        compiler_params=_params,
        scratch_types=[
            pltpu.VMEM((2 * K2CHUNK,), jnp.int32),
            pltpu.VMEM((2, K2CHUNK, PANEL), jnp.float32),
            pltpu.VMEM((2, K2CHUNK, PANEL), jnp.float32),
            pltpu.VMEM((DIM, 1024), jnp.float32),
            pltpu.VMEM((B_PER_W,), jnp.float32),
            [pltpu.SemaphoreType.DMA] * 2,
            pltpu.SemaphoreType.DMA,
        ],
    )
    return score(relation, hm, emb_rt)
